# Initial kernel scaffold; baseline (speedup 1.0000x reference)
import jax
import jax.numpy as jnp
from jax.experimental import pallas as pl


def kernel(sequence_output, W, b):
    raise NotImplementedError("write your pallas kernel here")


# speedup vs baseline: 55.0360x; 55.0360x over previous
import functools

import jax
import jax.numpy as jnp
from jax import lax
from jax.experimental import pallas as pl
from jax.experimental.pallas import tpu as pltpu
from jax.experimental.pallas import tpu_sc as plsc

B, S, H = 4, 2048, 4096

NC = 2
NS = 16
NW = NC * NS

HSPLIT = NW // B
HPW = H // HSPLIT
NLANE = 16
NGRP = HPW // NLANE

SCHUNK = 64
NCHUNK = S // SCHUNK
NPAIR = NCHUNK // 2


def _top3_body(x_hbm, out_hbm, buf0, buf1, tbuf, sem0, sem1):
  cid = lax.axis_index("c")
  sid = lax.axis_index("s")
  wid = sid * NC + cid
  bb = wid // HSPLIT
  h0 = (wid % HSPLIT) * HPW

  neg_inf = jnp.full((NLANE,), -jnp.inf, jnp.float32)
  for g in range(NGRP):
    sl = pl.ds(g * NLANE, NLANE)
    tbuf[0, sl] = neg_inf
    tbuf[1, sl] = neg_inf
    tbuf[2, sl] = neg_inf

  def start(c, buf, sem):
    pltpu.async_copy(
        x_hbm.at[bb, pl.ds(c * SCHUNK, SCHUNK), pl.ds(h0, HPW)], buf, sem)

  def wait(buf, sem):
    pltpu.make_async_copy(
        x_hbm.at[bb, pl.ds(0, SCHUNK), pl.ds(h0, HPW)], buf, sem).wait()

  def process(buf):
    @pl.loop(0, NGRP)
    def _(g):
      sl = pl.ds(g * NLANE, NLANE)
      t1 = tbuf[0, sl]
      t2 = tbuf[1, sl]
      t3 = tbuf[2, sl]
      for s in range(SCHUNK):
        v = buf[s, sl]
        d1 = jnp.minimum(t1, v)
        t1 = jnp.maximum(t1, v)
        d2 = jnp.minimum(t2, d1)
        t2 = jnp.maximum(t2, d1)
        t3 = jnp.maximum(t3, d2)
      tbuf[0, sl] = t1
      tbuf[1, sl] = t2
      tbuf[2, sl] = t3

  start(0, buf0, sem0)
  start(1, buf1, sem1)

  @pl.loop(0, NPAIR)
  def _(i):
    c0 = i * 2
    wait(buf0, sem0)
    process(buf0)

    @pl.when(c0 + 2 < NCHUNK)
    def _():
      start(c0 + 2, buf0, sem0)

    wait(buf1, sem1)
    process(buf1)

    @pl.when(c0 + 3 < NCHUNK)
    def _():
      start(c0 + 3, buf1, sem1)

  pltpu.sync_copy(tbuf, out_hbm.at[bb, :, pl.ds(h0, HPW)])


_top3 = functools.partial(
    pl.kernel,
    out_type=jax.ShapeDtypeStruct((B, 3, H), jnp.float32),
    mesh=plsc.VectorSubcoreMesh(core_axis_name="c", subcore_axis_name="s"),
    scratch_types=[
        pltpu.VMEM((SCHUNK, HPW), jnp.float32),
        pltpu.VMEM((SCHUNK, HPW), jnp.float32),
        pltpu.VMEM((3, HPW), jnp.float32),
        pltpu.SemaphoreType.DMA,
        pltpu.SemaphoreType.DMA,
    ],
)(_top3_body)


MPAD = 16
HBLK = 512


def _linear_body(p_ref, w_ref, b_ref, o_ref):
  acc = lax.dot_general(
      p_ref[...], w_ref[...], (((1,), (1,)), ((), ())),
      preferred_element_type=jnp.float32)
  o_ref[...] = jnp.tanh(acc + b_ref[...])


def _linear(p16, W, b2d):
  return pl.pallas_call(
      _linear_body,
      grid=(H // HBLK,),
      in_specs=[
          pl.BlockSpec((MPAD, H), lambda j: (0, 0)),
          pl.BlockSpec((HBLK, H), lambda j: (j, 0)),
          pl.BlockSpec((1, HBLK), lambda j: (0, j)),
      ],
      out_specs=pl.BlockSpec((MPAD, HBLK), lambda j: (0, j)),
      out_shape=jax.ShapeDtypeStruct((MPAD, H), jnp.float32),
  )(p16, W, b2d)


@jax.jit
def kernel(sequence_output, W, b):
  pooled = _top3(sequence_output)
  p16 = jnp.pad(pooled.reshape(B * 3, H), ((0, MPAD - B * 3), (0, 0)))
  out16 = _linear(p16, W, b.reshape(1, H))
  return out16[:B * 3].reshape(B, 3, H)


# speedup vs baseline: 55.8935x; 1.0156x over previous
import functools

import jax
import jax.numpy as jnp
from jax import lax
from jax.experimental import pallas as pl
from jax.experimental.pallas import tpu as pltpu
from jax.experimental.pallas import tpu_sc as plsc

B, S, H = 4, 2048, 4096

NC = 2
NS = 16
NW = NC * NS

HSPLIT = NW // B
HPW = H // HSPLIT
NLANE = 16
NGRP = HPW // NLANE

SCHUNK = 64
NCHUNK = S // SCHUNK
NPAIR = NCHUNK // 2


def _top3_body(x_hbm, out_hbm, buf0, buf1, tbuf, sem0, sem1):
  cid = lax.axis_index("c")
  sid = lax.axis_index("s")
  wid = sid * NC + cid
  bb = wid // HSPLIT
  h0 = (wid % HSPLIT) * HPW

  neg_inf = jnp.full((NLANE,), -jnp.inf, jnp.float32)
  for g in range(NGRP):
    sl = pl.ds(g * NLANE, NLANE)
    tbuf[0, sl] = neg_inf
    tbuf[1, sl] = neg_inf
    tbuf[2, sl] = neg_inf

  def start(c, buf, sem):
    pltpu.async_copy(
        x_hbm.at[bb, pl.ds(c * SCHUNK, SCHUNK), pl.ds(h0, HPW)], buf, sem)

  def wait(buf, sem):
    pltpu.make_async_copy(
        x_hbm.at[bb, pl.ds(0, SCHUNK), pl.ds(h0, HPW)], buf, sem).wait()

  def process(buf):
    @pl.loop(0, NGRP)
    def _(g):
      sl = pl.ds(g * NLANE, NLANE)
      t1 = tbuf[0, sl]
      t2 = tbuf[1, sl]
      t3 = tbuf[2, sl]
      for s in range(0, SCHUNK, 2):
        va = buf[s, sl]
        vb = buf[s + 1, sl]
        hi = jnp.maximum(va, vb)
        lo = jnp.minimum(va, vb)
        q = jnp.minimum(t1, hi)
        t1 = jnp.maximum(t1, hi)
        r = jnp.maximum(t2, lo)
        t2 = jnp.maximum(q, r)
        t3 = jnp.maximum(t3, jnp.minimum(q, r))
      tbuf[0, sl] = t1
      tbuf[1, sl] = t2
      tbuf[2, sl] = t3

  start(0, buf0, sem0)
  start(1, buf1, sem1)

  @pl.loop(0, NPAIR)
  def _(i):
    c0 = i * 2
    wait(buf0, sem0)
    process(buf0)

    @pl.when(c0 + 2 < NCHUNK)
    def _():
      start(c0 + 2, buf0, sem0)

    wait(buf1, sem1)
    process(buf1)

    @pl.when(c0 + 3 < NCHUNK)
    def _():
      start(c0 + 3, buf1, sem1)

  pltpu.sync_copy(tbuf, out_hbm.at[bb, :, pl.ds(h0, HPW)])


_top3 = functools.partial(
    pl.kernel,
    out_type=jax.ShapeDtypeStruct((B, 3, H), jnp.float32),
    mesh=plsc.VectorSubcoreMesh(core_axis_name="c", subcore_axis_name="s"),
    scratch_types=[
        pltpu.VMEM((SCHUNK, HPW), jnp.float32),
        pltpu.VMEM((SCHUNK, HPW), jnp.float32),
        pltpu.VMEM((3, HPW), jnp.float32),
        pltpu.SemaphoreType.DMA,
        pltpu.SemaphoreType.DMA,
    ],
)(_top3_body)


MPAD = 16
HBLK = 512


def _linear_body(p_ref, w_ref, b_ref, o_ref):
  acc = lax.dot_general(
      p_ref[...], w_ref[...], (((1,), (1,)), ((), ())),
      preferred_element_type=jnp.float32)
  o_ref[...] = jnp.tanh(acc + b_ref[...])


def _linear(p16, W, b2d):
  return pl.pallas_call(
      _linear_body,
      grid=(H // HBLK,),
      in_specs=[
          pl.BlockSpec((MPAD, H), lambda j: (0, 0)),
          pl.BlockSpec((HBLK, H), lambda j: (j, 0)),
          pl.BlockSpec((1, HBLK), lambda j: (0, j)),
      ],
      out_specs=pl.BlockSpec((MPAD, HBLK), lambda j: (0, j)),
      out_shape=jax.ShapeDtypeStruct((MPAD, H), jnp.float32),
  )(p16, W, b2d)


@jax.jit
def kernel(sequence_output, W, b):
  pooled = _top3(sequence_output)
  p16 = jnp.pad(pooled.reshape(B * 3, H), ((0, MPAD - B * 3), (0, 0)))
  out16 = _linear(p16, W, b.reshape(1, H))
  return out16[:B * 3].reshape(B, 3, H)
